# Initial kernel scaffold; baseline (speedup 1.0000x reference)
import jax
import jax.numpy as jnp
from jax.experimental import pallas as pl


def kernel(images, predictions):
    raise NotImplementedError("write your pallas kernel here")


# speedup vs baseline: 3.3089x; 3.3089x over previous
import functools

import jax
import jax.numpy as jnp
import numpy as np
from jax.experimental import pallas as pl

NUM_CLASSES = 80
CONF_THRESH = 0.05
NMS_IOU = 0.5
MAX_DET_PER_CLASS = 100
MAX_DET = 100
BOX_VARIANCE = np.array([0.1, 0.1, 0.2, 0.2], dtype=np.float32)


def _anchors_np(image_h, image_w):
    areas = [x ** 2 for x in [32.0, 64.0, 128.0, 256.0, 512.0]]
    scales = [2 ** x for x in [0.0, 1.0 / 3.0, 2.0 / 3.0]]
    aspect_ratios = [0.5, 1.0, 2.0]
    dims_all = []
    for area in areas:
        dims = []
        for ratio in aspect_ratios:
            h = np.sqrt(area / ratio)
            w = area / h
            for s in scales:
                dims.append(np.array([s * w, s * h], dtype=np.float32))
        dims_all.append(np.stack(dims, axis=0))
    strides = [2 ** i for i in range(3, 8)]
    anchors = []
    for lvl in range(5):
        fh = int(np.ceil(image_h / strides[lvl]))
        fw = int(np.ceil(image_w / strides[lvl]))
        rx = (np.arange(fw, dtype=np.float32) + 0.5) * strides[lvl]
        ry = (np.arange(fh, dtype=np.float32) + 0.5) * strides[lvl]
        cx, cy = np.meshgrid(rx, ry)
        centers = np.stack([cx, cy], axis=-1)
        centers = np.tile(centers[:, :, None, :], [1, 1, 9, 1])
        dims = np.tile(dims_all[lvl][None, None, :, :], [fh, fw, 1, 1])
        a = np.concatenate([centers, dims], axis=-1).reshape(-1, 4)
        anchors.append(a)
    return np.concatenate(anchors, axis=0)


def _decode_kernel(box_ref, anc_ref, cls_ref, dec_ref, sig_ref):
    a = anc_ref[...]
    xy = box_ref[:, :2] * 0.1 * a[:, 2:] + a[:, :2]
    wh = jnp.exp(box_ref[:, 2:] * 0.2) * a[:, 2:]
    dec_ref[...] = jnp.concatenate([xy - wh / 2.0, xy + wh / 2.0], axis=1)
    sig_ref[...] = jax.nn.sigmoid(cls_ref[...])


def _iou_matrix(boxes_a, boxes_b):
    lt = jnp.maximum(boxes_a[:, None, :2], boxes_b[None, :, :2])
    rb = jnp.minimum(boxes_a[:, None, 2:], boxes_b[None, :, 2:])
    wh = jnp.clip(rb - lt, 0.0)
    inter = wh[..., 0] * wh[..., 1]
    area_a = jnp.clip(boxes_a[:, 2] - boxes_a[:, 0], 0.0) * jnp.clip(boxes_a[:, 3] - boxes_a[:, 1], 0.0)
    area_b = jnp.clip(boxes_b[:, 2] - boxes_b[:, 0], 0.0) * jnp.clip(boxes_b[:, 3] - boxes_b[:, 1], 0.0)
    union = area_a[:, None] + area_b[None, :] - inter
    return inter / jnp.maximum(union, 1e-8)


def _nms_one(boxes, scores):
    k = MAX_DET_PER_CLASS
    top_scores, top_idx = jax.lax.top_k(scores, k)
    top_boxes = boxes[top_idx]
    iou = _iou_matrix(top_boxes, top_boxes)
    valid_init = top_scores > CONF_THRESH

    def body(i, keep):
        row = iou[i]
        suppress = (row > NMS_IOU) & (jnp.arange(k) > i) & keep[i]
        return keep & (~suppress)

    keep = jax.lax.fori_loop(0, k, body, valid_init)
    out_scores = jnp.where(keep, top_scores, -1.0)
    return top_boxes, out_scores


def kernel(images, predictions):
    N = predictions.shape[1]
    anchors = jnp.asarray(_anchors_np(images.shape[1], images.shape[2]))
    preds = predictions[0]
    box_preds = preds[:, :4]
    cls_logits = preds[:, 4:]

    BLK = 2048
    grid = (N + BLK - 1) // BLK
    decoded, sig = pl.pallas_call(
        _decode_kernel,
        grid=(grid,),
        in_specs=[
            pl.BlockSpec((BLK, 4), lambda i: (i, 0)),
            pl.BlockSpec((BLK, 4), lambda i: (i, 0)),
            pl.BlockSpec((BLK, NUM_CLASSES), lambda i: (i, 0)),
        ],
        out_specs=[
            pl.BlockSpec((BLK, 4), lambda i: (i, 0)),
            pl.BlockSpec((BLK, NUM_CLASSES), lambda i: (i, 0)),
        ],
        out_shape=[
            jax.ShapeDtypeStruct((N, 4), jnp.float32),
            jax.ShapeDtypeStruct((N, NUM_CLASSES), jnp.float32),
        ],
    )(box_preds, anchors, cls_logits)

    def per_class(c):
        return _nms_one(decoded, sig[:, c])

    all_boxes, all_scores = jax.vmap(per_class)(jnp.arange(NUM_CLASSES))
    all_classes = jnp.broadcast_to(jnp.arange(NUM_CLASSES)[:, None], all_scores.shape)
    flat_boxes = all_boxes.reshape(-1, 4)
    flat_scores = all_scores.reshape(-1)
    flat_classes = all_classes.reshape(-1)
    final_scores, final_idx = jax.lax.top_k(flat_scores, MAX_DET)
    final_boxes = flat_boxes[final_idx]
    final_classes = flat_classes[final_idx]
    valid = jnp.sum((final_scores > CONF_THRESH).astype(jnp.int32))
    return (final_boxes[None], final_scores[None], final_classes[None], valid[None])
